# baseline (speedup 1.0000x reference)
import functools
import jax
import jax.numpy as jnp
from jax import lax
from jax.experimental import pallas as pl
from jax.experimental.pallas import tpu as pltpu
from jax.experimental.pallas import tpu_sc as plsc

K = 1_000_000
D = 64
B = 16_384

FV_ROWS = K * D // 128
NEW_FV_ROWS = B * D // 128
NW = 32
FV_U = FV_ROWS // 8
SPAN = 15_632
OLD0 = SPAN - NEW_FV_ROWS

NEW_L = B
OLD_L_U = (K - B) // 8
LSPAN = 30_744


def _sc_body(nv_ref, fv_ref, ln_ref, lv_ref, out_f_ref, out_l_ref, lbuf, sem):
    wid = lax.axis_index("s") * 2 + lax.axis_index("c")
    base = jnp.minimum(8 * ((wid * FV_U) // NW), FV_ROWS - SPAN)
    base = pl.multiple_of(base, 8)

    @pl.when(wid == 0)
    def _():
        pltpu.make_async_copy(
            nv_ref, out_f_ref.at[pl.ds(0, NEW_FV_ROWS)], sem.at[0]).start()
        pltpu.make_async_copy(
            fv_ref.at[pl.ds(NEW_FV_ROWS, OLD0)],
            out_f_ref.at[pl.ds(NEW_FV_ROWS, OLD0)], sem.at[1]).start()
        pass

    @pl.when(wid != 0)
    def _():
        pltpu.make_async_copy(
            fv_ref.at[pl.ds(base, SPAN)],
            out_f_ref.at[pl.ds(base, SPAN)], sem.at[0]).start()

    l0 = jnp.minimum(NEW_L + 8 * ((wid * OLD_L_U) // NW), K - LSPAN)
    l0 = pl.multiple_of(l0, 8)
    pltpu.sync_copy(lv_ref.at[pl.ds(l0, LSPAN)], lbuf)
    pltpu.sync_copy(lbuf, out_l_ref.at[pl.ds(l0, LSPAN)])

    @pl.when(wid == 0)
    def _():
        pltpu.make_async_copy(
            nv_ref, out_f_ref.at[pl.ds(0, NEW_FV_ROWS)], sem.at[0]).wait()
        pltpu.make_async_copy(
            fv_ref.at[pl.ds(NEW_FV_ROWS, OLD0)],
            out_f_ref.at[pl.ds(NEW_FV_ROWS, OLD0)], sem.at[1]).wait()
        pltpu.sync_copy(ln_ref, lbuf.at[pl.ds(0, NEW_L)])
        pltpu.sync_copy(lbuf.at[pl.ds(0, NEW_L)],
                        out_l_ref.at[pl.ds(0, NEW_L)])

    @pl.when(wid != 0)
    def _():
        pltpu.make_async_copy(
            fv_ref.at[pl.ds(base, SPAN)],
            out_f_ref.at[pl.ds(base, SPAN)], sem.at[0]).wait()


def _make_sc_call():
    mesh = plsc.VectorSubcoreMesh(core_axis_name="c", subcore_axis_name="s")
    return functools.partial(
        pl.kernel,
        out_type=[
            jax.ShapeDtypeStruct((FV_ROWS, 128), jnp.float32),
            jax.ShapeDtypeStruct((K,), jnp.int32),
        ],
        mesh=mesh,
        scratch_types=[
            pltpu.VMEM((LSPAN,), jnp.int32),
            pltpu.SemaphoreType.DMA((4,)),
        ],
    )(_sc_body)


def kernel(feats, labels, features, labels_buf):
    fv = features.reshape(FV_ROWS, 128)
    nv = feats.reshape(NEW_FV_ROWS, 128)

    out_f, out_l = _make_sc_call()(nv, fv, labels, labels_buf)

    new_features = out_f.reshape(K, D)
    new_labels = out_l
    new_ptr = jnp.full((1,), B % K, dtype=jnp.int32)
    return (new_features, new_labels, new_ptr)


# speedup vs baseline: 6.4948x; 6.4948x over previous
import functools
import jax
import jax.numpy as jnp
from jax import lax
from jax.experimental import pallas as pl
from jax.experimental.pallas import tpu as pltpu
from jax.experimental.pallas import tpu_sc as plsc

K = 1_000_000
D = 64
B = 16_384

FV_ROWS = K * D // 128
NEW_FV_ROWS = B * D // 128
NW = 32
FV_U = FV_ROWS // 8
SPAN = 15_632
CH = 256
NSLOT = 3
NCH = 62
LAST_OFF = SPAN - CH

NEW_L = B
OLD_L_U = (K - B) // 8
LSPAN = 30_744
LC0 = 16_384
LC1 = LSPAN - LC0


def _sc_body(nv_ref, fv_ref, ln_ref, lv_ref, out_f_ref, out_l_ref,
             shbuf, lbuf, in_sem, out_sem, lsem):
    sid = lax.axis_index("s")
    wid = sid * 2 + lax.axis_index("c")
    fbuf = shbuf.at[sid]
    base = jnp.minimum(8 * ((wid * FV_U) // NW), FV_ROWS - SPAN)
    base = pl.multiple_of(base, 8)

    def chunk_start(c):
        return pl.multiple_of(base + jnp.minimum(c * CH, LAST_OFF), 8)

    def start_in(c):
        slot = lax.rem(c, NSLOT)
        g = chunk_start(c)

        @pl.when(g < NEW_FV_ROWS)
        def _():
            pltpu.make_async_copy(
                nv_ref.at[pl.ds(g, CH)], fbuf.at[slot], in_sem.at[slot]).start()

        @pl.when(g >= NEW_FV_ROWS)
        def _():
            pltpu.make_async_copy(
                fv_ref.at[pl.ds(g, CH)], fbuf.at[slot], in_sem.at[slot]).start()

    def wait_in(c):
        slot = lax.rem(c, NSLOT)
        pltpu.make_async_copy(
            fv_ref.at[pl.ds(0, CH)], fbuf.at[slot], in_sem.at[slot]).wait()

    def start_out(c):
        slot = lax.rem(c, NSLOT)
        g = chunk_start(c)
        pltpu.make_async_copy(
            fbuf.at[slot], out_f_ref.at[pl.ds(g, CH)], out_sem.at[slot]).start()

    def wait_out(c):
        slot = lax.rem(c, NSLOT)
        g = chunk_start(c)
        pltpu.make_async_copy(
            fbuf.at[slot], out_f_ref.at[pl.ds(g, CH)], out_sem.at[slot]).wait()

    l0 = jnp.minimum(NEW_L + 8 * ((wid * OLD_L_U) // NW), K - LSPAN)
    l0 = pl.multiple_of(l0, 8)
    l1 = pl.multiple_of(l0 + LC0, 8)
    bufA = lbuf.at[pl.ds(0, LC0)]
    bufB = lbuf.at[pl.ds(LC0, LC1)]
    in_A = lambda: pltpu.make_async_copy(
        lv_ref.at[pl.ds(l0, LC0)], bufA, lsem.at[0])
    out_A = lambda: pltpu.make_async_copy(
        bufA, out_l_ref.at[pl.ds(l0, LC0)], lsem.at[1])
    in_B = lambda: pltpu.make_async_copy(
        lv_ref.at[pl.ds(l1, LC1)], bufB, lsem.at[2])
    out_B = lambda: pltpu.make_async_copy(
        bufB, out_l_ref.at[pl.ds(l1, LC1)], lsem.at[3])
    in_N = lambda: pltpu.make_async_copy(ln_ref, bufA, lsem.at[0])
    out_N = lambda: pltpu.make_async_copy(
        bufA, out_l_ref.at[pl.ds(0, NEW_L)], lsem.at[1])

    start_in(0)
    in_A().start()
    in_B().start()

    def loop_body(c, carry):
        wait_in(c)

        @pl.when(c >= NSLOT - 1)
        def _():
            wait_out(c - (NSLOT - 1))

        @pl.when(c + 1 < NCH)
        def _():
            start_in(c + 1)
        start_out(c)

        @pl.when(c == 10)
        def _():
            in_A().wait()
            out_A().start()
            in_B().wait()
            out_B().start()

        @pl.when(jnp.logical_and(c == 30, wid == 0))
        def _():
            out_A().wait()
            in_N().start()

        @pl.when(jnp.logical_and(c == 40, wid == 0))
        def _():
            in_N().wait()
            out_N().start()
        return carry

    lax.fori_loop(0, NCH, loop_body, 0)
    wait_out(NCH - 2)
    wait_out(NCH - 1)

    @pl.when(wid != 0)
    def _():
        out_A().wait()
    out_B().wait()

    @pl.when(wid == 0)
    def _():
        out_N().wait()


def _make_sc_call():
    mesh = plsc.VectorSubcoreMesh(core_axis_name="c", subcore_axis_name="s")
    return functools.partial(
        pl.kernel,
        out_type=[
            jax.ShapeDtypeStruct((FV_ROWS, 128), jnp.float32),
            jax.ShapeDtypeStruct((K,), jnp.int32),
        ],
        mesh=mesh,
        scratch_types=[
            pltpu.VMEM_SHARED((16, NSLOT, CH, 128), jnp.float32),
            pltpu.VMEM((LSPAN,), jnp.int32),
            pltpu.SemaphoreType.DMA((NSLOT,)),
            pltpu.SemaphoreType.DMA((NSLOT,)),
            pltpu.SemaphoreType.DMA((4,)),
        ],
    )(_sc_body)


def kernel(feats, labels, features, labels_buf):
    fv = features.reshape(FV_ROWS, 128)
    nv = feats.reshape(NEW_FV_ROWS, 128)

    out_f, out_l = _make_sc_call()(nv, fv, labels, labels_buf)

    new_features = out_f.reshape(K, D)
    new_labels = out_l
    new_ptr = jnp.full((1,), B % K, dtype=jnp.int32)
    return (new_features, new_labels, new_ptr)
